# Initial kernel scaffold; baseline (speedup 1.0000x reference)
import jax
import jax.numpy as jnp
from jax.experimental import pallas as pl


def kernel(tnear, tfar, origins, dirs, dnorm, grid, u_rand):
    raise NotImplementedError("write your pallas kernel here")


# speedup vs baseline: 21.4462x; 21.4462x over previous
import jax
import jax.numpy as jnp
from jax import lax
from jax.experimental import pallas as pl
from jax.experimental.pallas import tpu as pltpu

_N = 131072
_K = 32
_S = 256
_G = 128
_RB = 256
_INTERPRET = False


def _shift_down(x, d, fill):
    pad = jnp.full((d,) + x.shape[1:], fill, x.dtype)
    return jnp.concatenate([pad, x[:-d]], axis=0)


def _sample_sort_body(scal_ref, dens_ref, u_ref, out_ref):
    tn = scal_ref[0:1, :]
    tf = scal_ref[1:2, :]
    dn = scal_ref[2:3, :]
    dens = dens_ref[...]
    u = u_ref[...]

    span = tf - tn
    delta = span * (dn * (1.0 / (_K - 1.0)))

    a = 1.0 - jnp.exp(-jnp.maximum(dens, 0.0) * delta)
    q = 1.0 - a + 1e-10
    p = _shift_down(q, 1, 1.0)
    for d in (1, 2, 4, 8, 16):
        p = p * _shift_down(p, d, 1.0)
    w = a * p

    riota = lax.broadcasted_iota(jnp.int32, (_K, 1), 0)
    r = jnp.where(riota < _K - 1, w + 1e-5, 0.0)
    total = jnp.sum(r, axis=0, keepdims=True)
    inv_total = 1.0 / total
    csum = r
    for d in (1, 2, 4, 8, 16):
        csum = csum + _shift_down(csum, d, 0.0)
    cdf = _shift_down(csum, 1, 0.0) * inv_total
    cdf_next = csum * inv_total

    jf = riota.astype(jnp.float32) * (1.0 / (_K - 1.0))
    ts = tn + span * jf
    tsn = jnp.concatenate([ts[1:], ts[-1:]], axis=0)
    denom = cdf_next - cdf
    denom = jnp.where(denom < 1e-5, 1.0, denom)
    slope = jnp.where(riota < _K - 1, (tsn - ts) / denom, 0.0)

    cb = jnp.broadcast_to(cdf[0:1], (_S, _RB))
    tb = jnp.broadcast_to(ts[0:1], (_S, _RB))
    sb = jnp.broadcast_to(slope[0:1], (_S, _RB))
    for j in range(1, _K):
        m = u >= cdf[j:j + 1]
        cb = jnp.where(m, cdf[j:j + 1], cb)
        tb = jnp.where(m, ts[j:j + 1], tb)
        sb = jnp.where(m, slope[j:j + 1], sb)
    x = tb + (u - cb) * sb

    p = 1
    while p < _S:
        k = p
        while k >= 1:
            if k == p:
                z = x.reshape(_S // (2 * p), 2, p, _RB)
                lo = z[:, 0]
                hi = z[:, 1]
                x = jnp.concatenate(
                    [jnp.minimum(lo, hi)[:, None], jnp.maximum(lo, hi)[:, None]],
                    axis=1).reshape(_S, _RB)
            else:
                b = x.reshape(_S // (2 * p), 2 * p, _RB)
                head = b[:, :k]
                tail = b[:, 2 * p - k:]
                mid = b[:, k:2 * p - k].reshape(
                    _S // (2 * p), (2 * p - 2 * k) // (2 * k), 2, k, _RB)
                lo = mid[:, :, 0]
                hi = mid[:, :, 1]
                mid2 = jnp.concatenate(
                    [jnp.minimum(lo, hi)[:, :, None],
                     jnp.maximum(lo, hi)[:, :, None]],
                    axis=2).reshape(_S // (2 * p), 2 * p - 2 * k, _RB)
                x = jnp.concatenate([head, mid2, tail], axis=1).reshape(_S, _RB)
            k //= 2
        p *= 2

    out_ref[...] = x


def _stage_b(scal, dens_t, u_t):
    return pl.pallas_call(
        _sample_sort_body,
        grid=(_N // _RB,),
        in_specs=[
            pl.BlockSpec((8, _RB), lambda i: (0, i)),
            pl.BlockSpec((_K, _RB), lambda i: (0, i)),
            pl.BlockSpec((_S, _RB), lambda i: (0, i)),
        ],
        out_specs=pl.BlockSpec((_S, _RB), lambda i: (0, i)),
        out_shape=jax.ShapeDtypeStruct((_S, _N), jnp.float32),
        interpret=_INTERPRET,
    )(scal, dens_t, u_t)


def _density_jnp(grid, tnear, tfar, origins, dirs):
    s = jnp.linspace(0.0, 1.0, _K)
    ts = tnear + (tfar - tnear) * s[None, :]
    pts = origins[:, None, :] + dirs[:, None, :] * ts[:, :, None]
    g = grid.shape[0]
    coords = (pts + 1.0) * 0.5 * (g - 1)
    coords = jnp.clip(coords, 0.0, g - 1 - 1e-6)
    c0 = jnp.floor(coords).astype(jnp.int32)
    c1 = jnp.minimum(c0 + 1, g - 1)
    f = coords - c0.astype(coords.dtype)
    x0, y0, z0 = c0[..., 0], c0[..., 1], c0[..., 2]
    x1, y1, z1 = c1[..., 0], c1[..., 1], c1[..., 2]
    fx, fy, fz = f[..., 0], f[..., 1], f[..., 2]
    v000 = grid[x0, y0, z0]
    v001 = grid[x0, y0, z1]
    v010 = grid[x0, y1, z0]
    v011 = grid[x0, y1, z1]
    v100 = grid[x1, y0, z0]
    v101 = grid[x1, y0, z1]
    v110 = grid[x1, y1, z0]
    v111 = grid[x1, y1, z1]
    v00 = v000 * (1 - fz) + v001 * fz
    v01 = v010 * (1 - fz) + v011 * fz
    v10 = v100 * (1 - fz) + v101 * fz
    v11 = v110 * (1 - fz) + v111 * fz
    v0 = v00 * (1 - fy) + v01 * fy
    v1 = v10 * (1 - fy) + v11 * fy
    return v0 * (1 - fx) + v1 * fx


def kernel(tnear, tfar, origins, dirs, dnorm, grid, u_rand):
    dens = _density_jnp(grid, tnear, tfar, origins, dirs)
    zeros = jnp.zeros((5, _N), jnp.float32)
    scal = jnp.concatenate(
        [tnear.T, tfar.T, dnorm.T, zeros], axis=0)
    out_t = _stage_b(scal, dens.T, u_rand.T)
    return out_t.T
